# baseline (speedup 1.0000x reference)
import functools

import jax
import jax.numpy as jnp
from jax import lax
from jax.experimental import pallas as pl
from jax.experimental.layout import Layout, with_layout_constraint
from jax.experimental.pallas import tpu as pltpu
from jax.experimental.pallas import tpu_sc as plsc

VOCAB = 1_000_000
EMBED = 32
OUT = 32
PACK = 4
WIDE = PACK * EMBED
TBLK = 10000
NW = 32
CH = 128
SCH = 2 * CH
L = 16


def _transform_body(t_ref, w_ref, b_ref, o_ref):
    o_ref[...] = (
        jnp.dot(t_ref[...], w_ref[...], preferred_element_type=jnp.float32)
        + b_ref[...]
    )


def _transform(table, W, b):
    packed = table.reshape(VOCAB // PACK, WIDE)
    bd = jnp.kron(jnp.eye(PACK, dtype=jnp.float32), W.T)
    bt = jnp.tile(b, PACK).reshape(1, WIDE)
    grid = (VOCAB // PACK) // TBLK
    return pl.pallas_call(
        _transform_body,
        grid=(grid,),
        in_specs=[
            pl.BlockSpec((TBLK, WIDE), lambda i: (i, 0)),
            pl.BlockSpec((WIDE, WIDE), lambda i: (0, 0)),
            pl.BlockSpec((1, WIDE), lambda i: (0, 0)),
        ],
        out_specs=pl.BlockSpec((TBLK, WIDE), lambda i: (i, 0)),
        out_shape=jax.ShapeDtypeStruct((VOCAB // PACK, WIDE), jnp.float32),
    )(packed, bd, bt)


def _gather(tbl4, idx):
    B = idx.shape[0]
    b_per_w = B // NW
    n_stages = b_per_w // SCH
    n_pairs = n_stages // 2
    mesh = plsc.VectorSubcoreMesh(core_axis_name="c", subcore_axis_name="s")

    @functools.partial(
        pl.kernel,
        mesh=mesh,
        out_type=jax.ShapeDtypeStruct((B * EMBED,), jnp.float32),
        compiler_params=pltpu.CompilerParams(needs_layout_passes=False),
        scratch_types=[
            pltpu.VMEM((b_per_w,), jnp.int32),
            [[pltpu.VMEM((CH,), jnp.int32)] * 2] * 2,
            [pltpu.VMEM((SCH, WIDE), jnp.float32)] * 2,
            [pltpu.VMEM((SCH * EMBED,), jnp.float32)] * 2,
            [pltpu.SemaphoreType.DMA] * 2,
            [pltpu.SemaphoreType.DMA] * 2,
        ],
    )
    def k(tbl_hbm, idx_hbm, out_hbm, idx_all, q_v, wide_v, out_v, sg, sw):
        wid = lax.axis_index("s") * 2 + lax.axis_index("c")
        base = wid * b_per_w
        pltpu.sync_copy(idx_hbm.at[pl.ds(base, b_per_w)], idx_all)

        def compute_q(s, buf):
            for h in range(2):
                for g in range(CH // L):
                    q_v[buf][h][pl.ds(g * L, L)] = lax.shift_right_logical(
                        idx_all[pl.ds(s * SCH + h * CH + g * L, L)], 2
                    )

        def fire_gather(buf):
            for h in range(2):
                pltpu.async_copy(
                    tbl_hbm.at[q_v[buf][h]],
                    wide_v[buf].at[pl.ds(h * CH, CH)],
                    sg[buf],
                )

        def wait_gather(buf):
            for h in range(2):
                pltpu.make_async_copy(
                    tbl_hbm.at[q_v[buf][h]],
                    wide_v[buf].at[pl.ds(h * CH, CH)],
                    sg[buf],
                ).wait()

        def extract(s, buf):
            for g in range(SCH // L):
                m = lax.bitwise_and(idx_all[pl.ds(s * SCH + g * L, L)], 3)
                colbase = lax.shift_left(m, 5)
                rows16 = g * L + lax.iota(jnp.int32, L)
                outbase = EMBED * rows16
                for j in range(EMBED):
                    vals = plsc.load_gather(wide_v[buf], [rows16, colbase + j])
                    plsc.store_scatter(out_v[buf], [outbase + j], vals)

        def fire_writeback(s, buf):
            off = (base + s * SCH) * EMBED
            pltpu.async_copy(out_v[buf], out_hbm.at[pl.ds(off, SCH * EMBED)], sw[buf])

        def drain_writeback(buf):
            pltpu.make_async_copy(
                out_hbm.at[pl.ds(base * EMBED, SCH * EMBED)], out_v[buf], sw[buf]
            ).wait()

        compute_q(0, 0)
        fire_gather(0)

        def body(p, carry):
            s0 = 2 * p
            s1 = s0 + 1
            compute_q(s1, 1)
            fire_gather(1)
            wait_gather(0)

            @pl.when(p != 0)
            def _():
                drain_writeback(0)

            extract(s0, 0)
            fire_writeback(s0, 0)

            @pl.when(p != n_pairs - 1)
            def _():
                compute_q(s0 + 2, 0)
                fire_gather(0)

            wait_gather(1)

            @pl.when(p != 0)
            def _():
                drain_writeback(1)

            extract(s1, 1)
            fire_writeback(s1, 1)
            return carry

        lax.fori_loop(0, n_pairs, body, 0)
        drain_writeback(0)
        drain_writeback(1)

    return k(tbl4, idx)


_COMPACT2 = Layout(major_to_minor=(0, 1), tiling=((8,),))
_COMPACT3 = Layout(major_to_minor=(0, 1, 2), tiling=((8,),))


def kernel(x, table, W, b):
    T4 = _transform(table, W, b)
    flat = x.reshape(-1).astype(jnp.int32)
    out = _gather(T4, flat)
    return out.reshape(x.shape + (OUT,))


# speedup vs baseline: 1.4243x; 1.4243x over previous
import functools

import jax
import jax.numpy as jnp
from jax import lax
from jax.experimental import pallas as pl
from jax.experimental.pallas import tpu as pltpu
from jax.experimental.pallas import tpu_sc as plsc

VOCAB = 1_000_000
EMBED = 32
OUT = 32
PACK = 4
WIDE = PACK * EMBED
NPACKED = VOCAB // PACK
TBLK = 10000
NW = 32
CH = 128
L = 16


def _transform_body(t_ref, w_ref, b_ref, o_ref):
    o_ref[0] = (
        jnp.dot(t_ref[...], w_ref[0], preferred_element_type=jnp.float32)
        + b_ref[...]
    )


def _transform(table, W, b):
    packed = table.reshape(NPACKED, WIDE)
    bd = jnp.kron(jnp.eye(PACK, dtype=jnp.float32), W.T)
    bd4 = jnp.stack([jnp.roll(bd, -EMBED * m, axis=1) for m in range(PACK)])
    bt = jnp.tile(b, PACK).reshape(1, WIDE)
    nblk = NPACKED // TBLK
    out = pl.pallas_call(
        _transform_body,
        grid=(nblk, PACK),
        in_specs=[
            pl.BlockSpec((TBLK, WIDE), lambda r, m: (r, 0)),
            pl.BlockSpec((1, WIDE, WIDE), lambda r, m: (m, 0, 0)),
            pl.BlockSpec((1, WIDE), lambda r, m: (0, 0)),
        ],
        out_specs=pl.BlockSpec((1, TBLK, WIDE), lambda r, m: (m, r, 0)),
        out_shape=jax.ShapeDtypeStruct((PACK, NPACKED, WIDE), jnp.float32),
    )(packed, bd4, bt)
    return out.reshape(VOCAB, WIDE)


def _gather(tblr, idx):
    B = idx.shape[0]
    b_per_w = B // NW
    n_stages = b_per_w // CH
    n_pairs = n_stages // 2
    mesh = plsc.VectorSubcoreMesh(core_axis_name="c", subcore_axis_name="s")

    @functools.partial(
        pl.kernel,
        mesh=mesh,
        out_type=jax.ShapeDtypeStruct((B * EMBED,), jnp.float32),
        compiler_params=pltpu.CompilerParams(needs_layout_passes=False),
        scratch_types=[
            pltpu.VMEM((b_per_w,), jnp.int32),
            [pltpu.VMEM((CH,), jnp.int32)] * 2,
            [pltpu.VMEM((CH, WIDE), jnp.float32)] * 2,
            [pltpu.VMEM((CH * EMBED,), jnp.float32)] * 2,
            [pltpu.SemaphoreType.DMA] * 2,
            [pltpu.SemaphoreType.DMA] * 2,
        ],
    )
    def k(tbl_hbm, idx_hbm, out_hbm, idx_all, q_v, wide_v, out_v, sg, sw):
        wid = lax.axis_index("s") * 2 + lax.axis_index("c")
        base = wid * b_per_w
        pltpu.sync_copy(idx_hbm.at[pl.ds(base, b_per_w)], idx_all)

        def compute_q(s, buf):
            for g in range(CH // L):
                v = idx_all[pl.ds(s * CH + g * L, L)]
                q_v[buf][pl.ds(g * L, L)] = (
                    lax.bitwise_and(v, 3) * NPACKED
                    + lax.shift_right_logical(v, 2)
                )

        def fire_gather(buf):
            pltpu.async_copy(tbl_hbm.at[q_v[buf]], wide_v[buf], sg[buf])

        def wait_gather(buf):
            pltpu.make_async_copy(tbl_hbm.at[q_v[buf]], wide_v[buf], sg[buf]).wait()

        def compact(buf):
            for r in range(CH):
                out_v[buf][pl.ds(r * EMBED, L)] = wide_v[buf][r, pl.ds(0, L)]
                out_v[buf][pl.ds(r * EMBED + L, L)] = wide_v[buf][r, pl.ds(L, L)]

        def fire_writeback(s, buf):
            off = (base + s * CH) * EMBED
            pltpu.async_copy(out_v[buf], out_hbm.at[pl.ds(off, CH * EMBED)], sw[buf])

        def drain_writeback(buf):
            pltpu.make_async_copy(
                out_hbm.at[pl.ds(base * EMBED, CH * EMBED)], out_v[buf], sw[buf]
            ).wait()

        compute_q(0, 0)
        fire_gather(0)

        def body(p, carry):
            s0 = 2 * p
            s1 = s0 + 1
            compute_q(s1, 1)
            fire_gather(1)
            wait_gather(0)

            @pl.when(p != 0)
            def _():
                drain_writeback(0)

            compact(0)
            fire_writeback(s0, 0)

            @pl.when(p != n_pairs - 1)
            def _():
                compute_q(s0 + 2, 0)
                fire_gather(0)

            wait_gather(1)

            @pl.when(p != 0)
            def _():
                drain_writeback(1)

            compact(1)
            fire_writeback(s1, 1)
            return carry

        lax.fori_loop(0, n_pairs, body, 0)
        drain_writeback(0)
        drain_writeback(1)

    return k(tblr, idx)


def kernel(x, table, W, b):
    Tr = _transform(table, W, b)
    flat = x.reshape(-1).astype(jnp.int32)
    out = _gather(Tr, flat)
    return out.reshape(x.shape + (OUT,))
